# baseline (speedup 1.0000x reference)
import functools

import jax
import jax.numpy as jnp
from jax import lax
from jax.experimental import pallas as pl
from jax.experimental.pallas import tpu as pltpu
from jax.experimental.pallas import tpu_sc as plsc

NUM_FIELDS = 26
VOCAB = 100000
HIDDEN = 128
BATCH = 16384

NC = 2
NS = 16
NW = NC * NS

TOTAL_ROWS = BATCH * NUM_FIELDS
ROWS_PER_W = TOTAL_ROWS // NW
CHUNK = 104
HALF_TILES = CHUNK // 8
NCHUNKS = ROWS_PER_W // CHUNK
NBUF = 8
NGROUPS = NCHUNKS // NBUF
STRIPES = BATCH // 8
STRIPES_PER_W = STRIPES // NW
SROWS = 8 * NUM_FIELDS


def _gather_body(table_hbm, idx_hbm, out_hbm, idx_v, *scratch):
    rows = scratch[:NBUF]
    gsem = scratch[NBUF:2 * NBUF]
    wsem = scratch[2 * NBUF:3 * NBUF]
    wid = lax.axis_index("s") * NC + lax.axis_index("c")
    pltpu.sync_copy(idx_hbm.at[wid], idx_v)
    sbase = wid * STRIPES_PER_W

    def start_gather(j, b):
        pltpu.async_copy(
            table_hbm.at[idx_v.at[j]], rows[b].reshape(CHUNK, HIDDEN),
            gsem[b],
        )

    def wait_gather(j, b):
        pltpu.make_async_copy(
            table_hbm.at[idx_v.at[j]], rows[b].reshape(CHUNK, HIDDEN),
            gsem[b],
        ).wait()

    def out_slice(j):
        return out_hbm.at[
            sbase + j // 2, pl.ds((j % 2) * HALF_TILES, HALF_TILES)
        ]

    def start_write(j, b):
        pltpu.async_copy(rows[b], out_slice(j), wsem[b])

    def wait_write(j, b):
        pltpu.make_async_copy(rows[b], out_slice(j), wsem[b]).wait()

    GB = NBUF // 2
    NGRP = NCHUNKS // GB

    def proc(g, par, first, last):
        s, o = par * GB, (1 - par) * GB
        for i in range(GB):
            wait_gather(g * GB + i, s + i)
        for i in range(GB):
            start_write(g * GB + i, s + i)
        if not first:
            for i in range(GB):
                wait_write((g - 1) * GB + i, o + i)
        if not last:
            for i in range(GB):
                start_gather((g + 1) * GB + i, o + i)

    for i in range(GB):
        start_gather(i, i)
    proc(0, 0, first=True, last=False)

    def pair_step(G, _):
        proc(2 * G + 1, 1, first=False, last=False)
        proc(2 * G + 2, 0, first=False, last=False)
        return ()

    lax.fori_loop(0, (NGRP - 2) // 2, pair_step, (), unroll=False)

    proc(NGRP - 1, 1, first=False, last=True)
    for i in range(GB):
        wait_write((NGRP - 1) * GB + i, GB + i)


@jax.jit
def _multi_embed(flat_idx, flat_table):
    mesh = plsc.VectorSubcoreMesh(
        core_axis_name="c", subcore_axis_name="s", num_cores=NC,
        num_subcores=NS,
    )
    run = pl.kernel(
        _gather_body,
        out_type=jax.ShapeDtypeStruct(
            (STRIPES, NUM_FIELDS, 8, HIDDEN), jnp.float32
        ),
        mesh=mesh,
        scratch_types=(
            [pltpu.VMEM((NCHUNKS, CHUNK), jnp.int32)]
            + [pltpu.VMEM((HALF_TILES, 8, HIDDEN), jnp.float32)] * NBUF
            + [pltpu.SemaphoreType.DMA] * (2 * NBUF)
        ),
    )
    return run(flat_table, flat_idx)


def kernel(x_n_cat, tables):
    flat = (
        x_n_cat.astype(jnp.int32)
        + jnp.arange(NUM_FIELDS, dtype=jnp.int32)[None, :] * VOCAB
    )
    perm = flat.reshape(NW, STRIPES_PER_W, 8, NUM_FIELDS).transpose(
        0, 1, 3, 2
    )
    flat_idx = perm.reshape(NW, NCHUNKS, CHUNK)
    flat_table = tables.reshape(NUM_FIELDS * VOCAB, HIDDEN)
    out4 = _multi_embed(flat_idx, flat_table)
    return out4.transpose(0, 2, 1, 3).reshape(BATCH, NUM_FIELDS * HIDDEN)


# speedup vs baseline: 1.0870x; 1.0870x over previous
import functools

import jax
import jax.numpy as jnp
from jax import lax
from jax.experimental import pallas as pl
from jax.experimental.pallas import tpu as pltpu
from jax.experimental.pallas import tpu_sc as plsc

NUM_FIELDS = 26
VOCAB = 100000
HIDDEN = 128
BATCH = 16384

NC = 2
NS = 16
NW = NC * NS

TOTAL_ROWS = BATCH * NUM_FIELDS
ROWS_PER_W = TOTAL_ROWS // NW
CHUNK = 104
HALF_TILES = CHUNK // 8
NCHUNKS = ROWS_PER_W // CHUNK
NBUF = 8
NGROUPS = NCHUNKS // NBUF
STRIPES = BATCH // 8
STRIPES_PER_W = STRIPES // NW
SROWS = 8 * NUM_FIELDS


def _gather_body(table_hbm, idx_hbm, out_hbm, idx_v, *scratch):
    rows = scratch[:NBUF]
    gsem = scratch[NBUF:2 * NBUF]
    wsem = scratch[2 * NBUF:3 * NBUF]
    wid = lax.axis_index("s") * NC + lax.axis_index("c")
    pltpu.sync_copy(idx_hbm.at[wid], idx_v)
    sbase = wid * STRIPES_PER_W

    def start_gather(j, b):
        pltpu.async_copy(
            table_hbm.at[idx_v.at[j]], rows[b].reshape(CHUNK, HIDDEN),
            gsem[b],
        )

    def wait_gather(j, b):
        pltpu.make_async_copy(
            table_hbm.at[idx_v.at[j]], rows[b].reshape(CHUNK, HIDDEN),
            gsem[b],
        ).wait()

    def out_slice(j):
        return out_hbm.at[
            sbase + j // 2, pl.ds((j % 2) * HALF_TILES, HALF_TILES)
        ]

    def start_write(j, b):
        pltpu.async_copy(rows[b], out_slice(j), wsem[b])

    def wait_write(j, b):
        pltpu.make_async_copy(rows[b], out_slice(j), wsem[b]).wait()

    for b in range(NBUF):
        start_gather(b, b)

    def group_step(g, _):
        j0 = g * NBUF
        for b in range(NBUF):
            wait_gather(j0 + b, b)
            start_write(j0 + b, b)
        for b in range(NBUF):
            wait_write(j0 + b, b)
            start_gather(j0 + NBUF + b, b)
        return ()

    lax.fori_loop(0, NGROUPS - 1, group_step, (), unroll=False)

    j0 = (NGROUPS - 1) * NBUF
    for b in range(NBUF):
        wait_gather(j0 + b, b)
        start_write(j0 + b, b)
    for b in range(NBUF):
        wait_write(j0 + b, b)


@jax.jit
def _multi_embed(flat_idx, flat_table):
    mesh = plsc.VectorSubcoreMesh(
        core_axis_name="c", subcore_axis_name="s", num_cores=NC,
        num_subcores=NS,
    )
    run = pl.kernel(
        _gather_body,
        out_type=jax.ShapeDtypeStruct(
            (STRIPES, NUM_FIELDS, 8, HIDDEN), jnp.float32
        ),
        mesh=mesh,
        scratch_types=(
            [pltpu.VMEM((NCHUNKS, CHUNK), jnp.int32)]
            + [pltpu.VMEM((HALF_TILES, 8, HIDDEN), jnp.float32)] * NBUF
            + [pltpu.SemaphoreType.DMA] * (2 * NBUF)
        ),
    )
    return run(flat_table, flat_idx)


def kernel(x_n_cat, tables):
    flat = (
        x_n_cat.astype(jnp.int32)
        + jnp.arange(NUM_FIELDS, dtype=jnp.int32)[None, :] * VOCAB
    )
    perm = flat.reshape(NW, STRIPES_PER_W, 8, NUM_FIELDS).transpose(
        0, 1, 3, 2
    )
    flat_idx = perm.reshape(NW, NCHUNKS, CHUNK)
    flat_table = tables.reshape(NUM_FIELDS * VOCAB, HIDDEN)
    out4 = _multi_embed(flat_idx, flat_table)
    return out4.transpose(0, 2, 1, 3).reshape(BATCH, NUM_FIELDS * HIDDEN)
